# baseline (speedup 1.0000x reference)
import functools

import jax
import jax.numpy as jnp
from jax import lax
from jax.experimental import pallas as pl
from jax.experimental.pallas import tpu as pltpu
from jax.experimental.pallas import tpu_sc as plsc

N = 10000
E = 160000
NHOP = 40000
R = 500
DIN = 128
RELD = 128
NHID = 64
HEADS = 2
OUT1 = 128
B = 4096
ALPHA = 0.2

NC = 2
NS = 16
NW = NC * NS
L = 16

C = 128
NP1 = 10016
RPAD = 512
TW = 136
HW = 72
ROWS_PER_TILE = NP1 // NS
MB = B // NW

NCR = 40
NCH = 10
TOT_CHUNKS = NW * (NCR + NCH)


def _norm_rows(x):
    n = jnp.sqrt(jnp.sum(x * x, axis=1, keepdims=True))
    return x / jnp.maximum(n, 1e-12)


def _elu(x):
    return jnp.where(x > 0, x, jnp.exp(jnp.minimum(x, 0.0)) - 1.0)


def _stage_a(ent_ref, relp_ref, a0t_ref, a1t_ref, a2t_ref, wu0_ref, wsp_ref,
             went_ref, a2ot_ref, wu0o_ref,
             p0_ref, p1x_ref, qx_ref, u0_ref, ew_ref, or1_ref, q2x_ref):
    ent = _norm_rows(ent_ref[...])
    dot = functools.partial(jnp.dot, preferred_element_type=jnp.float32)
    p0 = dot(ent, a0t_ref[...])
    p1 = dot(ent, a1t_ref[...])
    p0_ref[...] = p0
    u0_ref[...] = dot(p0, wu0_ref[...])
    p1x_ref[...] = jnp.concatenate([p1, dot(p1, wu0_ref[...])], axis=1)
    ew_ref[...] = dot(ent, went_ref[...])

    @pl.when(pl.program_id(0) == 0)
    def _rel_side():
        rel = _norm_rows(relp_ref[...])
        q = dot(rel, a2t_ref[...])
        qx_ref[...] = jnp.concatenate([q, dot(q, wu0_ref[...])], axis=1)
        or1 = dot(rel, wsp_ref[...])
        or1_ref[...] = or1
        q2 = dot(or1, a2ot_ref[...])
        q2x_ref[...] = jnp.concatenate([q2, dot(q2, wu0o_ref[...])], axis=1)


def _stage_c(p0_ref, acc_ref, rs_ref, a0ot_ref, a1ot_ref, wu0o_ref,
             p0o_ref, p1xo_ref, u0o_ref):
    a = acc_ref[...]
    acc = jnp.concatenate([a[0, 0] + a[0, 1], a[1, 0] + a[1, 1]], axis=1)
    rs = rs_ref[0] + rs_ref[1]
    p0 = p0_ref[...]
    xs = []
    for h in range(HEADS):
        rsh = rs[:, h:h + 1]
        hh = p0[:, h * NHID:(h + 1) * NHID] + acc[:, h * NHID:(h + 1) * NHID] / jnp.where(rsh > 0, rsh, 1.0)
        xs.append(_elu(jnp.where(rsh > 0, hh, 0.0)))
    x = jnp.concatenate(xs, axis=1)
    dot = functools.partial(jnp.dot, preferred_element_type=jnp.float32)
    p0o = dot(x, a0ot_ref[...])
    p1o = dot(x, a1ot_ref[...])
    p0o_ref[...] = p0o
    u0o_ref[...] = dot(p0o, wu0o_ref[...])
    p1xo_ref[...] = jnp.concatenate([p1o, dot(p1o, wu0o_ref[...])], axis=1)


def _stage_e(ew_ref, p0o_ref, acc_ref, rs_ref, out_ref):
    a = acc_ref[...]
    acc = jnp.concatenate([a[0, 0] + a[0, 1], a[1, 0] + a[1, 1]], axis=1)
    rsall = rs_ref[0] + rs_ref[1]
    rs = rsall[:, 0:1]
    h2 = p0o_ref[...] + acc / jnp.where(rs > 0, rs, 1.0)
    x2 = _elu(jnp.where(rs > 0, h2, 0.0))
    mask = jnp.where(rsall[:, 4:5] > 0, 1.0, 0.0)
    out_ref[...] = _norm_rows(ew_ref[...] + mask * x2)


def _make_edge_kernel(nheads, with_mask):
    mesh = plsc.VectorSubcoreMesh(core_axis_name="c", subcore_axis_name="s")
    out_type = [
        jax.ShapeDtypeStruct((2, NC, NP1, 64), jnp.float32),
        jax.ShapeDtypeStruct((NC, NP1, 8), jnp.float32),
    ]
    scratch = [
        pltpu.VMEM((2, 4, C), jnp.int32),
        pltpu.VMEM((C, 8), jnp.float32),
        pltpu.VMEM((C, 8), jnp.float32),
        pltpu.VMEM((C, HW), jnp.float32),
        pltpu.VMEM((C, HW), jnp.float32),
        pltpu.VMEM((C, 64), jnp.float32),
        pltpu.VMEM((C, 64), jnp.float32),
        pltpu.VMEM((C, 8), jnp.float32),
        pltpu.VMEM((C, 8), jnp.float32),
        pltpu.VMEM_SHARED((NP1, 64), jnp.float32),
        pltpu.VMEM_SHARED((NP1, 8), jnp.float32),
    ] + [pltpu.SemaphoreType.DMA] * 8
    if with_mask:
        scratch += [
            pltpu.VMEM((MB,), jnp.int32),
            pltpu.VMEM((MB, 8), jnp.float32),
        ]

    def body(idx4_hbm, u0_hbm, p1lo_hbm, p1hi_hbm, qlo_hbm, qhi_hbm,
             zn_hbm, zr_hbm, zw_hbm, ones_hbm, *rest):
        if with_mask:
            (midx_hbm, acc_out, rs_out,
             idxP, u0g0, u0g1, p1g0, p1g1, yg0, yg1, wrows0, wrows1,
             accsh, rssh,
             su0, su1, sp0, sp1, sf0, sf1, sw0, sw1,
             midxv, onesv) = rest
        else:
            (acc_out, rs_out,
             idxP, u0g0, u0g1, p1g0, p1g1, yg0, yg1, wrows0, wrows1,
             accsh, rssh,
             su0, su1, sp0, sp1, sf0, sf1, sw0, sw1) = rest
        cid = lax.axis_index("c")
        sid = lax.axis_index("s")
        wid = cid * NS + sid
        rb = sid * ROWS_PER_TILE
        lanes = lax.iota(jnp.int32, L)

        u0g = (u0g0, u0g1)
        p1g = (p1g0, p1g1)
        yg = (yg0, yg1)
        wrows = (wrows0, wrows1)
        semu = (su0, su1)
        semp = (sp0, sp1)
        semf = (sf0, sf1)
        semw = (sw0, sw1)

        pltpu.sync_copy(zr_hbm.at[pl.ds(rb, ROWS_PER_TILE)],
                        rssh.at[pl.ds(rb, ROWS_PER_TILE)])
        if with_mask:
            pltpu.sync_copy(ones_hbm, onesv)

        def zero_acc():
            pltpu.sync_copy(zn_hbm.at[pl.ds(rb, ROWS_PER_TILE)],
                            accsh.at[pl.ds(rb, ROWS_PER_TILE)])

        def run_pass(p, p1t_hbm, qt_hbm):
            hp = p if nheads == 2 else 0
            add_rs = (nheads == 2) or (p == 0)
            col_u = jnp.full((L,), 64 + hp, jnp.int32)
            col_u0 = jnp.full((L,), hp, jnp.int32)
            pltpu.sync_copy(zw_hbm, wrows[0])
            pltpu.sync_copy(zw_hbm, wrows[1])

            def drain_flush(b):
                pltpu.make_async_copy(zn_hbm.at[pl.ds(0, C)], yg[b], semf[b]).wait()
                if add_rs:
                    pltpu.make_async_copy(zr_hbm.at[pl.ds(0, C)], wrows[b], semw[b]).wait()

            def fire(b, jc, k):
                @pl.when(k > 0)
                def _():
                    drain_flush(b)
                pltpu.sync_copy(idx4_hbm.at[jc], idxP.at[b])
                cu = pltpu.async_copy(u0_hbm.at[idxP.at[b, 0]], u0g[b], semu[b])
                cp = pltpu.async_copy(p1t_hbm.at[idxP.at[b, 1]], p1g[b], semp[b])
                return (cu, cp)

            def fireq(b, copies):
                for cc in copies:
                    cc.wait()
                cq0 = pltpu.async_copy(qt_hbm.at[idxP.at[b, 2]], p1g[b],
                                       semp[b], add=True)
                cq1 = pltpu.async_copy(qt_hbm.at[idxP.at[b, 3]], p1g[b],
                                       semu[b], add=True)
                return (cq0, cq1)

            def compute_flush(b, copies):
                for cc in copies:
                    cc.wait()

                @plsc.parallel_loop(0, C // L, unroll=2)
                def group(g):
                    idx = g * L + lanes
                    s = (plsc.load_gather(u0g[b], [idx, col_u0])
                         + plsc.load_gather(p1g[b], [idx, col_u]))
                    w = jnp.exp(-(jnp.maximum(s, 0.0)
                                  + ALPHA * jnp.minimum(s, 0.0)))
                    if add_rs:
                        plsc.store_scatter(
                            wrows[b], [idx, jnp.full((L,), hp, jnp.int32)], w)
                    for f in range(64):
                        bf = jnp.full((L,), f, jnp.int32)
                        v = plsc.load_gather(p1g[b], [idx, bf])
                        plsc.store_scatter(yg[b], [idx, bf], w * v)
                pltpu.async_copy(yg[b], accsh.at[idxP.at[b, 0]], semf[b], add=True)
                if add_rs:
                    pltpu.async_copy(wrows[b], rssh.at[idxP.at[b, 0]], semw[b], add=True)

            cbase = wid * (NCR + NCH)

            def pair(k, _):
                jc = cbase + 2 * k
                c0 = fire(0, jc, k)
                c1 = fire(1, jc + 1, k)
                q0 = fireq(0, c0)
                q1 = fireq(1, c1)
                compute_flush(0, q0)
                compute_flush(1, q1)
                return 0
            lax.fori_loop(0, (NCR + NCH) // 2, pair, 0)
            drain_flush(0)
            drain_flush(1)

        zero_acc()
        plsc.subcore_barrier()
        if with_mask:
            pltpu.sync_copy(midx_hbm.at[pl.ds(wid * MB, MB)], midxv)
            pltpu.sync_copy(onesv, rssh.at[midxv], add=True)
        run_pass(0, p1lo_hbm, qlo_hbm)
        plsc.subcore_barrier()
        pltpu.sync_copy(accsh.at[pl.ds(rb, ROWS_PER_TILE)],
                        acc_out.at[0, cid, pl.ds(rb, ROWS_PER_TILE)])
        plsc.subcore_barrier()
        zero_acc()
        plsc.subcore_barrier()
        run_pass(1, p1hi_hbm, qhi_hbm)
        plsc.subcore_barrier()
        pltpu.sync_copy(accsh.at[pl.ds(rb, ROWS_PER_TILE)],
                        acc_out.at[1, cid, pl.ds(rb, ROWS_PER_TILE)])
        pltpu.sync_copy(rssh.at[pl.ds(rb, ROWS_PER_TILE)],
                        rs_out.at[cid, pl.ds(rb, ROWS_PER_TILE)])

    return pl.kernel(body, out_type=out_type, mesh=mesh, scratch_types=scratch,
                     compiler_params=pltpu.CompilerParams(
                         use_tc_tiling_on_sc=False, needs_layout_passes=False))


_edge_l1 = _make_edge_kernel(HEADS, False)
_edge_l2 = _make_edge_kernel(1, True)


def _pad_rows(x, rows):
    return jnp.pad(x, ((0, rows - x.shape[0]), (0, 0)))


def _halves(t):
    lo = jnp.concatenate([t[:, 0:64], t[:, 128:136]], axis=1)
    hi = jnp.concatenate([t[:, 64:128], t[:, 128:136]], axis=1)
    return lo, hi


def _region_idx(dst, src, t0, t1, per_w, padded_w, fill_t):
    def shape1(x, fill):
        x = x.reshape(NW, per_w)
        x = jnp.pad(x, ((0, 0), (0, padded_w - per_w)), constant_values=fill)
        return x.reshape(NW, padded_w // C, C)
    return jnp.stack([shape1(dst, N), shape1(src, N),
                      shape1(t0, fill_t), shape1(t1, fill_t)],
                     axis=2)


def kernel(Corpus_, batch_inputs, edge_list, edge_type, train_indices_nhop,
           entity_embeddings, relation_embeddings, W_entities, W_spgat,
           a_heads, a2_heads, a_out, a2_out):
    f32 = jnp.float32
    i32 = jnp.int32

    a0t = jnp.concatenate([a_heads[0, :, 0:DIN].T, a_heads[1, :, 0:DIN].T], axis=1)
    a1t = jnp.concatenate([a_heads[0, :, DIN:2 * DIN].T, a_heads[1, :, DIN:2 * DIN].T], axis=1)
    a2t = jnp.concatenate([a_heads[0, :, 2 * DIN:].T, a_heads[1, :, 2 * DIN:].T], axis=1)
    wu0 = jnp.zeros((128, 8), f32)
    wu0 = wu0.at[0:NHID, 0].set(a2_heads[0, 0])
    wu0 = wu0.at[NHID:128, 1].set(a2_heads[1, 0])
    a0ot = a_out[:, 0:OUT1].T
    a1ot = a_out[:, OUT1:2 * OUT1].T
    a2ot = a_out[:, 2 * OUT1:].T
    wu0o = jnp.zeros((128, 8), f32).at[:, 0].set(a2_out[0])

    relp = _pad_rows(relation_embeddings.astype(f32), RPAD)

    BN = 2000
    NG = N // BN
    _row = lambda w: pl.BlockSpec((BN, w), lambda i: (i, 0))
    _full = lambda r, w: pl.BlockSpec((r, w), lambda i: (0, 0))
    p0, p1x, qx, u0, ew, or1p, q2x = pl.pallas_call(
        _stage_a,
        grid=(NG,),
        in_specs=[_row(128), _full(RPAD, 128), _full(128, 128), _full(128, 128),
                  _full(128, 128), _full(128, 8), _full(128, 128),
                  _full(128, 128), _full(128, 128), _full(128, 8)],
        out_specs=[_row(128), _row(TW), _full(RPAD, TW), _row(8), _row(128),
                   _full(RPAD, 128), _full(RPAD, TW)],
        out_shape=[
            jax.ShapeDtypeStruct((N, 128), f32),
            jax.ShapeDtypeStruct((N, TW), f32),
            jax.ShapeDtypeStruct((RPAD, TW), f32),
            jax.ShapeDtypeStruct((N, 8), f32),
            jax.ShapeDtypeStruct((N, 128), f32),
            jax.ShapeDtypeStruct((RPAD, 128), f32),
            jax.ShapeDtypeStruct((RPAD, TW), f32),
        ],
    )(entity_embeddings.astype(f32), relp, a0t, a1t, a2t, wu0,
      W_spgat.astype(f32), W_entities.astype(f32), a2ot, wu0o)

    nh = train_indices_nhop.astype(i32)
    idx_reg = _region_idx(edge_list[0].astype(i32), edge_list[1].astype(i32),
                          edge_type.astype(i32), jnp.full((E,), R, i32),
                          E // NW, NCR * C, R)
    idx_nh = _region_idx(nh[:, 3], nh[:, 0], nh[:, 1], nh[:, 2],
                         NHOP // NW, NCH * C, R)
    idx4 = jnp.concatenate([idx_reg, idx_nh], axis=1).reshape(-1, 4, C)

    zn = jnp.zeros((NP1, 64), f32)
    zr = jnp.zeros((NP1, 8), f32)
    zw = jnp.zeros((C, 8), f32)
    ones = jnp.zeros((MB, 8), f32).at[:, 4].set(1.0)

    p1lo, p1hi = _halves(_pad_rows(p1x, NP1))
    qlo, qhi = _halves(qx)
    acc1, rs1 = _edge_l1(idx4, _pad_rows(u0, NP1), p1lo, p1hi, qlo, qhi,
                         zn, zr, zw, ones)

    _row4 = pl.BlockSpec((2, NC, BN, 64), lambda i: (0, 0, i, 0))
    _row3 = lambda w: pl.BlockSpec((NC, BN, w), lambda i: (0, i, 0))
    p0o, p1xo, u0o = pl.pallas_call(
        _stage_c,
        grid=(NG,),
        in_specs=[_row(128), _row4, _row3(8),
                  _full(128, 128), _full(128, 128), _full(128, 8)],
        out_specs=[_row(128), _row(TW), _row(8)],
        out_shape=[
            jax.ShapeDtypeStruct((N, 128), f32),
            jax.ShapeDtypeStruct((N, TW), f32),
            jax.ShapeDtypeStruct((N, 8), f32),
        ],
    )(p0, acc1[:, :, :N, :], rs1[:, :N, :], a0ot, a1ot, wu0o)

    midx = batch_inputs[:, 2].astype(i32)
    p1lo2, p1hi2 = _halves(_pad_rows(p1xo, NP1))
    qlo2, qhi2 = _halves(q2x)
    acc2, rs2 = _edge_l2(idx4, _pad_rows(u0o, NP1), p1lo2, p1hi2, qlo2, qhi2,
                         zn, zr, zw, ones, midx)

    out1 = pl.pallas_call(
        _stage_e,
        grid=(NG,),
        in_specs=[_row(128), _row(128), _row4, _row3(8)],
        out_specs=_row(128),
        out_shape=jax.ShapeDtypeStruct((N, 128), f32),
    )(ew, p0o, acc2[:, :, :N, :], rs2[:, :N, :])

    return (out1, or1p[:R])


# speedup vs baseline: 7.6084x; 7.6084x over previous
import functools

import jax
import jax.numpy as jnp
from jax import lax
from jax.experimental import pallas as pl
from jax.experimental.pallas import tpu as pltpu
from jax.experimental.pallas import tpu_sc as plsc

N = 10000
E = 160000
NHOP = 40000
R = 500
DIN = 128
RELD = 128
NHID = 64
HEADS = 2
OUT1 = 128
B = 4096
ALPHA = 0.2

NC = 2
NS = 16
NW = NC * NS
L = 16

C = 128
NP1 = 10016
RPAD = 512
TW = 136
HW = 72
ROWS_PER_TILE = NP1 // NS
MB = B // NW

NCR = 40
NCH = 10
TOT_CHUNKS = NW * (NCR + NCH)


def _norm_rows(x):
    n = jnp.sqrt(jnp.sum(x * x, axis=1, keepdims=True))
    return x / jnp.maximum(n, 1e-12)


def _elu(x):
    return jnp.where(x > 0, x, jnp.exp(jnp.minimum(x, 0.0)) - 1.0)


def _stage_a(ent_ref, relp_ref, a0t_ref, a1t_ref, a2t_ref, wu0_ref, wsp_ref,
             went_ref, a2ot_ref, wu0o_ref,
             p0_ref, p1x_ref, qx_ref, u0_ref, ew_ref, or1_ref, q2x_ref):
    ent = _norm_rows(ent_ref[...])
    dot = functools.partial(jnp.dot, preferred_element_type=jnp.float32)
    p0 = dot(ent, a0t_ref[...])
    p1 = dot(ent, a1t_ref[...])
    p0_ref[...] = p0
    u0_ref[...] = dot(p0, wu0_ref[...])
    p1x_ref[...] = jnp.concatenate([p1, dot(p1, wu0_ref[...])], axis=1)
    ew_ref[...] = dot(ent, went_ref[...])

    @pl.when(pl.program_id(0) == 0)
    def _rel_side():
        rel = _norm_rows(relp_ref[...])
        q = dot(rel, a2t_ref[...])
        qx_ref[...] = jnp.concatenate([q, dot(q, wu0_ref[...])], axis=1)
        or1 = dot(rel, wsp_ref[...])
        or1_ref[...] = or1
        q2 = dot(or1, a2ot_ref[...])
        q2x_ref[...] = jnp.concatenate([q2, dot(q2, wu0o_ref[...])], axis=1)


def _stage_c(p0_ref, acc_ref, rs_ref, a0ot_ref, a1ot_ref, wu0o_ref,
             p0o_ref, p1xo_ref, u0o_ref):
    a = acc_ref[...]
    acc = jnp.concatenate([a[0, 0] + a[0, 1], a[1, 0] + a[1, 1]], axis=1)
    rs = rs_ref[0] + rs_ref[1]
    p0 = p0_ref[...]
    xs = []
    for h in range(HEADS):
        rsh = rs[:, h:h + 1]
        hh = p0[:, h * NHID:(h + 1) * NHID] + acc[:, h * NHID:(h + 1) * NHID] / jnp.where(rsh > 0, rsh, 1.0)
        xs.append(_elu(jnp.where(rsh > 0, hh, 0.0)))
    x = jnp.concatenate(xs, axis=1)
    dot = functools.partial(jnp.dot, preferred_element_type=jnp.float32)
    p0o = dot(x, a0ot_ref[...])
    p1o = dot(x, a1ot_ref[...])
    p0o_ref[...] = p0o
    u0o_ref[...] = dot(p0o, wu0o_ref[...])
    p1xo_ref[...] = jnp.concatenate([p1o, dot(p1o, wu0o_ref[...])], axis=1)


def _stage_e(ew_ref, p0o_ref, acc_ref, rs_ref, out_ref):
    a = acc_ref[...]
    acc = jnp.concatenate([a[0, 0] + a[0, 1], a[1, 0] + a[1, 1]], axis=1)
    rsall = rs_ref[0] + rs_ref[1]
    rs = rsall[:, 0:1]
    h2 = p0o_ref[...] + acc / jnp.where(rs > 0, rs, 1.0)
    x2 = _elu(jnp.where(rs > 0, h2, 0.0))
    mask = jnp.where(rsall[:, 4:5] > 0, 1.0, 0.0)
    out_ref[...] = _norm_rows(ew_ref[...] + mask * x2)


def _make_edge_kernel(nheads, with_mask):
    mesh = plsc.VectorSubcoreMesh(core_axis_name="c", subcore_axis_name="s")
    out_type = [
        jax.ShapeDtypeStruct((2, NC, NP1, 64), jnp.float32),
        jax.ShapeDtypeStruct((NC, NP1, 8), jnp.float32),
    ]
    scratch = [
        pltpu.VMEM((2, 4, C), jnp.int32),
        pltpu.VMEM((C, 8), jnp.float32),
        pltpu.VMEM((C, 8), jnp.float32),
        pltpu.VMEM((C, HW), jnp.float32),
        pltpu.VMEM((C, HW), jnp.float32),
        pltpu.VMEM((C, 64), jnp.float32),
        pltpu.VMEM((C, 64), jnp.float32),
        pltpu.VMEM((C, 8), jnp.float32),
        pltpu.VMEM((C, 8), jnp.float32),
        pltpu.VMEM((RPAD, HW), jnp.float32),
        pltpu.VMEM_SHARED((NP1, 64), jnp.float32),
        pltpu.VMEM_SHARED((NP1, 8), jnp.float32),
    ] + [pltpu.SemaphoreType.DMA] * 8
    if with_mask:
        scratch += [
            pltpu.VMEM((MB,), jnp.int32),
            pltpu.VMEM((MB, 8), jnp.float32),
        ]

    def body(idx4_hbm, u0_hbm, p1lo_hbm, p1hi_hbm, qlo_hbm, qhi_hbm,
             zn_hbm, zr_hbm, zw_hbm, ones_hbm, *rest):
        if with_mask:
            (midx_hbm, acc_out, rs_out,
             idxP, u0g0, u0g1, p1g0, p1g1, yg0, yg1, wrows0, wrows1, qtab,
             accsh, rssh,
             su0, su1, sp0, sp1, sf0, sf1, sw0, sw1,
             midxv, onesv) = rest
        else:
            (acc_out, rs_out,
             idxP, u0g0, u0g1, p1g0, p1g1, yg0, yg1, wrows0, wrows1, qtab,
             accsh, rssh,
             su0, su1, sp0, sp1, sf0, sf1, sw0, sw1) = rest
        cid = lax.axis_index("c")
        sid = lax.axis_index("s")
        wid = cid * NS + sid
        rb = sid * ROWS_PER_TILE
        lanes = lax.iota(jnp.int32, L)

        u0g = (u0g0, u0g1)
        p1g = (p1g0, p1g1)
        yg = (yg0, yg1)
        wrows = (wrows0, wrows1)
        semu = (su0, su1)
        semp = (sp0, sp1)
        semf = (sf0, sf1)
        semw = (sw0, sw1)

        pltpu.sync_copy(zr_hbm.at[pl.ds(rb, ROWS_PER_TILE)],
                        rssh.at[pl.ds(rb, ROWS_PER_TILE)])
        if with_mask:
            pltpu.sync_copy(ones_hbm, onesv)

        def zero_acc():
            pltpu.sync_copy(zn_hbm.at[pl.ds(rb, ROWS_PER_TILE)],
                            accsh.at[pl.ds(rb, ROWS_PER_TILE)])

        def run_pass(p, p1t_hbm, qt_hbm):
            hp = p if nheads == 2 else 0
            add_rs = (nheads == 2) or (p == 0)
            col_u = jnp.full((L,), 64 + hp, jnp.int32)
            col_u0 = jnp.full((L,), hp, jnp.int32)
            pltpu.sync_copy(zw_hbm, wrows[0])
            pltpu.sync_copy(zw_hbm, wrows[1])
            pltpu.sync_copy(qt_hbm, qtab)

            def drain_flush(b):
                pltpu.make_async_copy(zn_hbm.at[pl.ds(0, C)], yg[b], semf[b]).wait()
                if add_rs:
                    pltpu.make_async_copy(zr_hbm.at[pl.ds(0, C)], wrows[b], semw[b]).wait()

            def fire(b, jc, k):
                @pl.when(k > 0)
                def _():
                    drain_flush(b)
                pltpu.sync_copy(idx4_hbm.at[jc], idxP.at[b])
                cu = pltpu.async_copy(u0_hbm.at[idxP.at[b, 0]], u0g[b], semu[b])
                cp = pltpu.async_copy(p1t_hbm.at[idxP.at[b, 1]], p1g[b], semp[b])
                return (cu, cp)

            def compute_flush(b, copies, has_q1):
                for cc in copies:
                    cc.wait()

                @plsc.parallel_loop(0, C // L, unroll=2)
                def group(g):
                    idx = g * L + lanes
                    t0v = idxP[b, 2, pl.ds(g * L, L)]
                    if has_q1:
                        t1v = idxP[b, 3, pl.ds(g * L, L)]
                    s = (plsc.load_gather(u0g[b], [idx, col_u0])
                         + plsc.load_gather(p1g[b], [idx, col_u])
                         + plsc.load_gather(qtab, [t0v, col_u]))
                    if has_q1:
                        s = s + plsc.load_gather(qtab, [t1v, col_u])
                    w = jnp.exp(-(jnp.maximum(s, 0.0)
                                  + ALPHA * jnp.minimum(s, 0.0)))
                    if add_rs:
                        plsc.store_scatter(
                            wrows[b], [idx, jnp.full((L,), hp, jnp.int32)], w)
                    for f in range(64):
                        bf = jnp.full((L,), f, jnp.int32)
                        v = (plsc.load_gather(p1g[b], [idx, bf])
                             + plsc.load_gather(qtab, [t0v, bf]))
                        if has_q1:
                            v = v + plsc.load_gather(qtab, [t1v, bf])
                        plsc.store_scatter(yg[b], [idx, bf], w * v)
                pltpu.async_copy(yg[b], accsh.at[idxP.at[b, 0]], semf[b], add=True)
                if add_rs:
                    pltpu.async_copy(wrows[b], rssh.at[idxP.at[b, 0]], semw[b], add=True)

            cbase = wid * (NCR + NCH)

            def pair_reg(k, _):
                jc = cbase + 2 * k
                c0 = fire(0, jc, k)
                c1 = fire(1, jc + 1, k)
                compute_flush(0, c0, False)
                compute_flush(1, c1, False)
                return 0
            lax.fori_loop(0, NCR // 2, pair_reg, 0)

            def pair_nh(k, _):
                jc = cbase + NCR + 2 * k
                c0 = fire(0, jc, k + 1)
                c1 = fire(1, jc + 1, k + 1)
                compute_flush(0, c0, True)
                compute_flush(1, c1, True)
                return 0
            lax.fori_loop(0, NCH // 2, pair_nh, 0)
            drain_flush(0)
            drain_flush(1)

        zero_acc()
        plsc.subcore_barrier()
        if with_mask:
            pltpu.sync_copy(midx_hbm.at[pl.ds(wid * MB, MB)], midxv)
            pltpu.sync_copy(onesv, rssh.at[midxv], add=True)
        run_pass(0, p1lo_hbm, qlo_hbm)
        plsc.subcore_barrier()
        pltpu.sync_copy(accsh.at[pl.ds(rb, ROWS_PER_TILE)],
                        acc_out.at[0, cid, pl.ds(rb, ROWS_PER_TILE)])
        plsc.subcore_barrier()
        zero_acc()
        plsc.subcore_barrier()
        run_pass(1, p1hi_hbm, qhi_hbm)
        plsc.subcore_barrier()
        pltpu.sync_copy(accsh.at[pl.ds(rb, ROWS_PER_TILE)],
                        acc_out.at[1, cid, pl.ds(rb, ROWS_PER_TILE)])
        pltpu.sync_copy(rssh.at[pl.ds(rb, ROWS_PER_TILE)],
                        rs_out.at[cid, pl.ds(rb, ROWS_PER_TILE)])

    return pl.kernel(body, out_type=out_type, mesh=mesh, scratch_types=scratch,
                     compiler_params=pltpu.CompilerParams(
                         use_tc_tiling_on_sc=False, needs_layout_passes=False))


_edge_l1 = _make_edge_kernel(HEADS, False)
_edge_l2 = _make_edge_kernel(1, True)


def _pad_rows(x, rows):
    return jnp.pad(x, ((0, rows - x.shape[0]), (0, 0)))


def _halves(t):
    lo = jnp.concatenate([t[:, 0:64], t[:, 128:136]], axis=1)
    hi = jnp.concatenate([t[:, 64:128], t[:, 128:136]], axis=1)
    return lo, hi


def _region_idx(dst, src, t0, t1, per_w, padded_w, fill_t):
    def shape1(x, fill):
        x = x.reshape(NW, per_w)
        x = jnp.pad(x, ((0, 0), (0, padded_w - per_w)), constant_values=fill)
        return x.reshape(NW, padded_w // C, C)
    return jnp.stack([shape1(dst, N), shape1(src, N),
                      shape1(t0, fill_t), shape1(t1, fill_t)],
                     axis=2)


def kernel(Corpus_, batch_inputs, edge_list, edge_type, train_indices_nhop,
           entity_embeddings, relation_embeddings, W_entities, W_spgat,
           a_heads, a2_heads, a_out, a2_out):
    f32 = jnp.float32
    i32 = jnp.int32

    a0t = jnp.concatenate([a_heads[0, :, 0:DIN].T, a_heads[1, :, 0:DIN].T], axis=1)
    a1t = jnp.concatenate([a_heads[0, :, DIN:2 * DIN].T, a_heads[1, :, DIN:2 * DIN].T], axis=1)
    a2t = jnp.concatenate([a_heads[0, :, 2 * DIN:].T, a_heads[1, :, 2 * DIN:].T], axis=1)
    wu0 = jnp.zeros((128, 8), f32)
    wu0 = wu0.at[0:NHID, 0].set(a2_heads[0, 0])
    wu0 = wu0.at[NHID:128, 1].set(a2_heads[1, 0])
    a0ot = a_out[:, 0:OUT1].T
    a1ot = a_out[:, OUT1:2 * OUT1].T
    a2ot = a_out[:, 2 * OUT1:].T
    wu0o = jnp.zeros((128, 8), f32).at[:, 0].set(a2_out[0])

    relp = _pad_rows(relation_embeddings.astype(f32), RPAD)

    BN = 2000
    NG = N // BN
    _row = lambda w: pl.BlockSpec((BN, w), lambda i: (i, 0))
    _full = lambda r, w: pl.BlockSpec((r, w), lambda i: (0, 0))
    p0, p1x, qx, u0, ew, or1p, q2x = pl.pallas_call(
        _stage_a,
        grid=(NG,),
        in_specs=[_row(128), _full(RPAD, 128), _full(128, 128), _full(128, 128),
                  _full(128, 128), _full(128, 8), _full(128, 128),
                  _full(128, 128), _full(128, 128), _full(128, 8)],
        out_specs=[_row(128), _row(TW), _full(RPAD, TW), _row(8), _row(128),
                   _full(RPAD, 128), _full(RPAD, TW)],
        out_shape=[
            jax.ShapeDtypeStruct((N, 128), f32),
            jax.ShapeDtypeStruct((N, TW), f32),
            jax.ShapeDtypeStruct((RPAD, TW), f32),
            jax.ShapeDtypeStruct((N, 8), f32),
            jax.ShapeDtypeStruct((N, 128), f32),
            jax.ShapeDtypeStruct((RPAD, 128), f32),
            jax.ShapeDtypeStruct((RPAD, TW), f32),
        ],
    )(entity_embeddings.astype(f32), relp, a0t, a1t, a2t, wu0,
      W_spgat.astype(f32), W_entities.astype(f32), a2ot, wu0o)

    nh = train_indices_nhop.astype(i32)
    idx_reg = _region_idx(edge_list[0].astype(i32), edge_list[1].astype(i32),
                          edge_type.astype(i32), jnp.full((E,), R, i32),
                          E // NW, NCR * C, R)
    idx_nh = _region_idx(nh[:, 3], nh[:, 0], nh[:, 1], nh[:, 2],
                         NHOP // NW, NCH * C, R)
    idx4 = jnp.concatenate([idx_reg, idx_nh], axis=1).reshape(-1, 4, C)

    zn = jnp.zeros((NP1, 64), f32)
    zr = jnp.zeros((NP1, 8), f32)
    zw = jnp.zeros((C, 8), f32)
    ones = jnp.zeros((MB, 8), f32).at[:, 4].set(1.0)

    p1lo, p1hi = _halves(_pad_rows(p1x, NP1))
    qlo, qhi = _halves(qx)
    acc1, rs1 = _edge_l1(idx4, _pad_rows(u0, NP1), p1lo, p1hi, qlo, qhi,
                         zn, zr, zw, ones)

    _row4 = pl.BlockSpec((2, NC, BN, 64), lambda i: (0, 0, i, 0))
    _row3 = lambda w: pl.BlockSpec((NC, BN, w), lambda i: (0, i, 0))
    p0o, p1xo, u0o = pl.pallas_call(
        _stage_c,
        grid=(NG,),
        in_specs=[_row(128), _row4, _row3(8),
                  _full(128, 128), _full(128, 128), _full(128, 8)],
        out_specs=[_row(128), _row(TW), _row(8)],
        out_shape=[
            jax.ShapeDtypeStruct((N, 128), f32),
            jax.ShapeDtypeStruct((N, TW), f32),
            jax.ShapeDtypeStruct((N, 8), f32),
        ],
    )(p0, acc1[:, :, :N, :], rs1[:, :N, :], a0ot, a1ot, wu0o)

    midx = batch_inputs[:, 2].astype(i32)
    p1lo2, p1hi2 = _halves(_pad_rows(p1xo, NP1))
    qlo2, qhi2 = _halves(q2x)
    acc2, rs2 = _edge_l2(idx4, _pad_rows(u0o, NP1), p1lo2, p1hi2, qlo2, qhi2,
                         zn, zr, zw, ones, midx)

    out1 = pl.pallas_call(
        _stage_e,
        grid=(NG,),
        in_specs=[_row(128), _row(128), _row4, _row3(8)],
        out_specs=_row(128),
        out_shape=jax.ShapeDtypeStruct((N, 128), f32),
    )(ew, p0o, acc2[:, :, :N, :], rs2[:, :N, :])

    return (out1, or1p[:R])
